# Initial kernel scaffold; baseline (speedup 1.0000x reference)
import jax
import jax.numpy as jnp
from jax.experimental import pallas as pl


def kernel(nf, edge_index, W_lin, W_attn):
    raise NotImplementedError("write your pallas kernel here")


# speedup vs baseline: 8.5739x; 8.5739x over previous
import functools

import jax
import jax.numpy as jnp
from jax import lax
from jax.experimental import pallas as pl
from jax.experimental.pallas import tpu as pltpu
from jax.experimental.pallas import tpu_sc as plsc

N = 10000
E = 160000
D = 256
WID = 144
HALF = 128
NS = 16
EPT = E // NS
CH = 80
NCH = EPT // CH
ACCR = 10240
SPT = ACCR // NS
NFULL = SPT // CH


def _stage_a(nf, W_attn):
    def body(nf_ref, wa_ref, out_ref):
        nfv = nf_ref[...]
        wa = wa_ref[0:D, 0]
        a = jnp.sum(nfv * wa[None, :], axis=1, keepdims=True)
        p = jnp.exp(a - jnp.max(a))
        out_ref[0:N, 0:HALF] = nfv[:, 0:HALF] * p
        out_ref[0:N, HALF:HALF + 1] = p
        out_ref[0:N, HALF + 1:WID] = jnp.zeros((N, WID - HALF - 1), jnp.float32)
        out_ref[N:2 * N, 0:HALF] = nfv[:, HALF:D] * p
        out_ref[N:2 * N, HALF:WID] = jnp.zeros((N, WID - HALF), jnp.float32)

    return pl.pallas_call(
        body,
        out_shape=jax.ShapeDtypeStruct((2 * N, WID), jnp.float32),
    )(nf, W_attn)


def _edge_stage(mst, src, dst):
    mesh = plsc.VectorSubcoreMesh(core_axis_name="c", subcore_axis_name="s")

    @functools.partial(
        pl.kernel,
        out_type=jax.ShapeDtypeStruct((2 * N, WID), jnp.float32),
        mesh=mesh,
        scratch_types=[
            pltpu.VMEM((CH,), jnp.int32),
            pltpu.VMEM((CH,), jnp.int32),
            pltpu.VMEM((CH, WID), jnp.float32),
            pltpu.VMEM_SHARED((ACCR, WID), jnp.float32),
            pltpu.SemaphoreType.DMA,
        ],
        compiler_params=pltpu.CompilerParams(use_tc_tiling_on_sc=False),
    )
    def run(mst_hbm, src_hbm, dst_hbm, out_hbm, idxs_v, idxd_v, rows_v, acc_sh, sem):
        c = lax.axis_index("c")
        s = lax.axis_index("s")

        def zrow(r, carry):
            for j in range(WID // 16):
                rows_v[r, pl.ds(j * 16, 16)] = jnp.zeros((16,), jnp.float32)
            return carry

        lax.fori_loop(0, CH, zrow, None)
        base_r = s * SPT
        for i in range(NFULL):
            pltpu.sync_copy(rows_v, acc_sh.at[pl.ds(base_r + i * CH, CH)])
        plsc.subcore_barrier()

        ebase = s * EPT
        coff = c * N

        def chunk(g, carry):
            eb = ebase + g * CH
            pltpu.sync_copy(src_hbm.at[pl.ds(eb, CH)], idxs_v)
            pltpu.sync_copy(dst_hbm.at[pl.ds(eb, CH)], idxd_v)
            for j in range(CH // 16):
                idxs_v[pl.ds(j * 16, 16)] = idxs_v[pl.ds(j * 16, 16)] + coff
            pltpu.async_copy(mst_hbm.at[idxs_v], rows_v, sem).wait()
            pltpu.sync_copy(rows_v, acc_sh.at[idxd_v], add=True)
            return carry

        lax.fori_loop(0, NCH, chunk, None)
        plsc.subcore_barrier()

        ob = coff + base_r
        nb = jnp.where(s == NS - 1, (N - (NS - 1) * SPT) // CH, NFULL)

        def oblk(i, carry):
            pltpu.sync_copy(acc_sh.at[pl.ds(base_r + i * CH, CH)], rows_v)
            pltpu.sync_copy(rows_v, out_hbm.at[pl.ds(ob + i * CH, CH)])
            return carry

        lax.fori_loop(0, nb, oblk, None)

    return run(mst, src, dst)


def _stage_c(rst, nf, W_lin):
    def body(rst_ref, nf_ref, wl_ref, out_ref):
        denom = rst_ref[0:N, HALF:HALF + 1]
        ok = denom > 0.0
        dsafe = jnp.where(ok, denom, 1.0)
        agg_lo = jnp.where(ok, rst_ref[0:N, 0:HALF] / dsafe, 0.0)
        agg_hi = jnp.where(ok, rst_ref[N:2 * N, 0:HALF] / dsafe, 0.0)
        agg = jnp.concatenate([agg_lo, agg_hi], axis=1)
        nfv = nf_ref[...]
        wl = wl_ref[...]
        acc = jnp.dot(nfv, wl[0:D, :], preferred_element_type=jnp.float32)
        acc = acc + jnp.dot(agg, wl[D:2 * D, :], preferred_element_type=jnp.float32)
        out_ref[...] = jnp.maximum(acc, 0.0)

    return pl.pallas_call(
        body,
        out_shape=jax.ShapeDtypeStruct((N, D), jnp.float32),
    )(rst, nf, W_lin)


def kernel(nf, edge_index, W_lin, W_attn):
    src = edge_index[0]
    dst = edge_index[1]
    mst = _stage_a(nf, W_attn)
    rst = _edge_stage(mst, src, dst)
    return _stage_c(rst, nf, W_lin)


# speedup vs baseline: 16.0123x; 1.8676x over previous
import functools

import jax
import jax.numpy as jnp
from jax import lax
from jax.experimental import pallas as pl
from jax.experimental.pallas import tpu as pltpu
from jax.experimental.pallas import tpu_sc as plsc

N = 10000
E = 160000
D = 256
WID = 144
HALF = 128
NS = 16
EPT = E // NS
CH = 80
NCH = EPT // CH
ACCR = 10240
SPT = ACCR // NS
NFULL = SPT // CH


def _stage_a(nf, W_attn):
    def body(nf_ref, wa_ref, out_ref):
        nfv = nf_ref[...]
        wa = wa_ref[0:D, 0]
        a = jnp.sum(nfv * wa[None, :], axis=1, keepdims=True)
        p = jnp.exp(a - jnp.max(a))
        out_ref[0:N, 0:HALF] = nfv[:, 0:HALF] * p
        out_ref[0:N, HALF:HALF + 1] = p
        out_ref[0:N, HALF + 1:WID] = jnp.zeros((N, WID - HALF - 1), jnp.float32)
        out_ref[N:2 * N, 0:HALF] = nfv[:, HALF:D] * p
        out_ref[N:2 * N, HALF:WID] = jnp.zeros((N, WID - HALF), jnp.float32)

    return pl.pallas_call(
        body,
        out_shape=jax.ShapeDtypeStruct((2 * N, WID), jnp.float32),
    )(nf, W_attn)


NT = NCH // 2


def _edge_stage(mst, src2, dst2):
    mesh = plsc.VectorSubcoreMesh(core_axis_name="c", subcore_axis_name="s")

    @functools.partial(
        pl.kernel,
        out_type=jax.ShapeDtypeStruct((2 * N, WID), jnp.float32),
        mesh=mesh,
        scratch_types=[
            [pltpu.VMEM((2, CH), jnp.int32)] * 2,
            [pltpu.VMEM((2, CH), jnp.int32)] * 2,
            [pltpu.VMEM((CH, WID), jnp.float32)] * 2,
            pltpu.VMEM_SHARED((ACCR, WID), jnp.float32),
            [pltpu.SemaphoreType.DMA] * 2,
            [pltpu.SemaphoreType.DMA] * 2,
            [pltpu.SemaphoreType.DMA] * 2,
            [pltpu.SemaphoreType.DMA] * 2,
        ],
        compiler_params=pltpu.CompilerParams(use_tc_tiling_on_sc=False),
    )
    def run(mst_hbm, src_hbm, dst_hbm, out_hbm, idxs, idxd, rows, acc_sh,
            semis, semid, semg, sems):
        c = lax.axis_index("c")
        s = lax.axis_index("s")
        coff = c * N
        irow0 = s * NCH

        def idx_start(t, r):
            base = irow0 + 2 * t
            pltpu.async_copy(src_hbm.at[pl.ds(base, 2)], idxs[r], semis[r])
            pltpu.async_copy(dst_hbm.at[pl.ds(base, 2)], idxd[r], semid[r])

        def idx_wait_bias(t, r):
            base = irow0 + 2 * t
            pltpu.make_async_copy(src_hbm.at[pl.ds(base, 2)], idxs[r], semis[r]).wait()
            pltpu.make_async_copy(dst_hbm.at[pl.ds(base, 2)], idxd[r], semid[r]).wait()
            for k in range(2):
                for j in range(CH // 16):
                    sl = pl.ds(j * 16, 16)
                    idxs[r][k, sl] = idxs[r][k, sl] + coff

        def gather_start(r, k, b):
            pltpu.async_copy(mst_hbm.at[idxs[r].at[k]], rows[b], semg[b])

        def gather_wait(r, k, b):
            pltpu.make_async_copy(mst_hbm.at[idxs[r].at[k]], rows[b], semg[b]).wait()

        def scat_start(r, k, b):
            pltpu.async_copy(rows[b], acc_sh.at[idxd[r].at[k]], sems[b], add=True)

        def scat_wait(r, k, b):
            pltpu.make_async_copy(rows[b], acc_sh.at[idxd[r].at[k]], sems[b]).wait()

        def zrow(r, carry):
            for j in range(WID // 16):
                rows[0][r, pl.ds(j * 16, 16)] = jnp.zeros((16,), jnp.float32)
            return carry

        lax.fori_loop(0, CH, zrow, None)
        base_r = s * SPT
        for i in range(NFULL):
            pltpu.sync_copy(rows[0], acc_sh.at[pl.ds(base_r + i * CH, CH)])
        plsc.subcore_barrier()

        idx_start(0, 0)
        idx_wait_bias(0, 0)
        gather_start(0, 0, 0)

        def turn(t, r, rn):
            @pl.when(t >= 1)
            def _():
                scat_wait(r, 1, 1)

            idx_start(t + 1, rn)
            gather_start(r, 1, 1)
            gather_wait(r, 0, 0)
            scat_start(r, 0, 0)
            idx_wait_bias(t + 1, rn)
            scat_wait(r, 0, 0)
            gather_start(rn, 0, 0)
            gather_wait(r, 1, 1)
            scat_start(r, 1, 1)
            return None

        def two_turns(u, carry):
            turn(2 * u, 0, 1)
            turn(2 * u + 1, 1, 0)
            return carry

        lax.fori_loop(0, NT // 2, two_turns, None)
        gather_wait(0, 0, 0)
        scat_start(0, 0, 0)
        scat_wait(1, 1, 1)
        scat_wait(0, 0, 0)
        plsc.subcore_barrier()

        ob = coff + base_r
        nb = jnp.where(s == NS - 1, (N - (NS - 1) * SPT) // CH, NFULL)

        def oblk(i, carry):
            pltpu.sync_copy(acc_sh.at[pl.ds(base_r + i * CH, CH)], rows[0])
            pltpu.sync_copy(rows[0], out_hbm.at[pl.ds(ob + i * CH, CH)])
            return carry

        lax.fori_loop(0, nb, oblk, None)

    return run(mst, src2, dst2)


def _stage_c(rst, nf, W_lin):
    def body(rst_ref, nf_ref, wl_ref, out_ref):
        denom = rst_ref[0:N, HALF:HALF + 1]
        ok = denom > 0.0
        dsafe = jnp.where(ok, denom, 1.0)
        agg_lo = jnp.where(ok, rst_ref[0:N, 0:HALF] / dsafe, 0.0)
        agg_hi = jnp.where(ok, rst_ref[N:2 * N, 0:HALF] / dsafe, 0.0)
        agg = jnp.concatenate([agg_lo, agg_hi], axis=1)
        nfv = nf_ref[...]
        wl = wl_ref[...]
        acc = jnp.dot(nfv, wl[0:D, :], preferred_element_type=jnp.float32)
        acc = acc + jnp.dot(agg, wl[D:2 * D, :], preferred_element_type=jnp.float32)
        out_ref[...] = jnp.maximum(acc, 0.0)

    return pl.pallas_call(
        body,
        out_shape=jax.ShapeDtypeStruct((N, D), jnp.float32),
    )(rst, nf, W_lin)


def kernel(nf, edge_index, W_lin, W_attn):
    src2 = jnp.pad(edge_index[0].reshape(E // CH, CH), ((0, 1), (0, 0)))
    dst2 = jnp.pad(edge_index[1].reshape(E // CH, CH), ((0, 1), (0, 0)))
    mst = _stage_a(nf, W_attn)
    rst = _edge_stage(mst, src2, dst2)
    return _stage_c(rst, nf, W_lin)


# speedup vs baseline: 16.1814x; 1.0106x over previous
import functools

import jax
import jax.numpy as jnp
from jax import lax
from jax.experimental import pallas as pl
from jax.experimental.pallas import tpu as pltpu
from jax.experimental.pallas import tpu_sc as plsc

N = 10000
E = 160000
D = 256
WID = 144
HALF = 128
NS = 16
EPT = E // NS
CH = 100
NCH = EPT // CH
ACCR = 10240
SPT = ACCR // NS
ZB = 80
NFULL = SPT // ZB


def _stage_a(nf, W_attn):
    def body(nf_ref, wa_ref, out_ref):
        nfv = nf_ref[...]
        wa = wa_ref[0:D, 0]
        a = jnp.sum(nfv * wa[None, :], axis=1, keepdims=True)
        p = jnp.exp(a - jnp.max(a))
        out_ref[0:N, 0:HALF] = nfv[:, 0:HALF] * p
        out_ref[0:N, HALF:HALF + 1] = p
        out_ref[0:N, HALF + 1:WID] = jnp.zeros((N, WID - HALF - 1), jnp.float32)
        out_ref[N:2 * N, 0:HALF] = nfv[:, HALF:D] * p
        out_ref[N:2 * N, HALF:WID] = jnp.zeros((N, WID - HALF), jnp.float32)

    return pl.pallas_call(
        body,
        out_shape=jax.ShapeDtypeStruct((2 * N, WID), jnp.float32),
    )(nf, W_attn)


NT = NCH // 2


def _edge_stage(mst, src2, dst2):
    mesh = plsc.VectorSubcoreMesh(core_axis_name="c", subcore_axis_name="s")

    @functools.partial(
        pl.kernel,
        out_type=jax.ShapeDtypeStruct((2 * N, WID), jnp.float32),
        mesh=mesh,
        scratch_types=[
            [pltpu.VMEM((2, CH), jnp.int32)] * 2,
            [pltpu.VMEM((2, CH), jnp.int32)] * 2,
            [pltpu.VMEM((CH, WID), jnp.float32)] * 2,
            pltpu.VMEM_SHARED((ACCR, WID), jnp.float32),
            [pltpu.SemaphoreType.DMA] * 2,
            [pltpu.SemaphoreType.DMA] * 2,
            [pltpu.SemaphoreType.DMA] * 2,
            [pltpu.SemaphoreType.DMA] * 2,
        ],
        compiler_params=pltpu.CompilerParams(use_tc_tiling_on_sc=False),
    )
    def run(mst_hbm, src_hbm, dst_hbm, out_hbm, idxs, idxd, rows, acc_sh,
            semis, semid, semg, sems):
        c = lax.axis_index("c")
        s = lax.axis_index("s")
        coff = c * N
        irow0 = s * NCH

        def idx_start(t, r):
            base = irow0 + 2 * t
            pltpu.async_copy(src_hbm.at[pl.ds(base, 2)], idxs[r], semis[r])
            pltpu.async_copy(dst_hbm.at[pl.ds(base, 2)], idxd[r], semid[r])

        def idx_wait_bias(t, r):
            base = irow0 + 2 * t
            pltpu.make_async_copy(src_hbm.at[pl.ds(base, 2)], idxs[r], semis[r]).wait()
            pltpu.make_async_copy(dst_hbm.at[pl.ds(base, 2)], idxd[r], semid[r]).wait()
            for k in range(2):
                for j in range(CH // 16):
                    sl = pl.ds(j * 16, 16)
                    idxs[r][k, sl] = idxs[r][k, sl] + coff
                rem = CH - (CH // 16) * 16
                if rem:
                    sl = pl.ds(CH - 16, 16)
                    lanes = lax.iota(jnp.int32, 16)
                    idxs[r][k, sl] = idxs[r][k, sl] + jnp.where(
                        lanes >= 16 - rem, coff, 0)

        def gather_start(r, k, b):
            pltpu.async_copy(mst_hbm.at[idxs[r].at[k]], rows[b], semg[b])

        def gather_wait(r, k, b):
            pltpu.make_async_copy(mst_hbm.at[idxs[r].at[k]], rows[b], semg[b]).wait()

        def scat_start(r, k, b):
            pltpu.async_copy(rows[b], acc_sh.at[idxd[r].at[k]], sems[b], add=True)

        def scat_wait(r, k, b):
            pltpu.make_async_copy(rows[b], acc_sh.at[idxd[r].at[k]], sems[b]).wait()

        def zrow(r, carry):
            for j in range(WID // 16):
                rows[0][r, pl.ds(j * 16, 16)] = jnp.zeros((16,), jnp.float32)
            return carry

        lax.fori_loop(0, ZB, zrow, None)
        base_r = s * SPT
        for i in range(NFULL):
            pltpu.sync_copy(rows[0].at[pl.ds(0, ZB)],
                            acc_sh.at[pl.ds(base_r + i * ZB, ZB)])
        plsc.subcore_barrier()

        idx_start(0, 0)
        idx_wait_bias(0, 0)
        gather_start(0, 0, 0)

        def turn(t, r, rn):
            @pl.when(t >= 1)
            def _():
                scat_wait(r, 1, 1)

            idx_start(t + 1, rn)
            gather_start(r, 1, 1)
            gather_wait(r, 0, 0)
            scat_start(r, 0, 0)
            idx_wait_bias(t + 1, rn)
            scat_wait(r, 0, 0)
            gather_start(rn, 0, 0)
            gather_wait(r, 1, 1)
            scat_start(r, 1, 1)
            return None

        def two_turns(u, carry):
            turn(2 * u, 0, 1)
            turn(2 * u + 1, 1, 0)
            return carry

        lax.fori_loop(0, NT // 2, two_turns, None)
        scat_wait(1, 1, 1)
        gather_wait(0, 0, 0)
        plsc.subcore_barrier()

        ob = coff + base_r
        nb = jnp.where(s == NS - 1, (N - (NS - 1) * SPT) // ZB, NFULL)

        def oblk(i, carry):
            pltpu.sync_copy(acc_sh.at[pl.ds(base_r + i * ZB, ZB)],
                            rows[0].at[pl.ds(0, ZB)])
            pltpu.sync_copy(rows[0].at[pl.ds(0, ZB)],
                            out_hbm.at[pl.ds(ob + i * ZB, ZB)])
            return carry

        lax.fori_loop(0, nb, oblk, None)

    return run(mst, src2, dst2)


CB = 1000


def _stage_c(rst, nf, W_lin):
    rst3 = rst.reshape(2, N, WID)

    def body(r0_ref, r1_ref, nf_ref, wl_ref, out_ref):
        denom = r0_ref[0, :, HALF:HALF + 1]
        ok = denom > 0.0
        dsafe = jnp.where(ok, denom, 1.0)
        agg_lo = jnp.where(ok, r0_ref[0, :, 0:HALF] / dsafe, 0.0)
        agg_hi = jnp.where(ok, r1_ref[0, :, 0:HALF] / dsafe, 0.0)
        agg = jnp.concatenate([agg_lo, agg_hi], axis=1).astype(jnp.bfloat16)
        nfv = nf_ref[...].astype(jnp.bfloat16)
        wl = wl_ref[...].astype(jnp.bfloat16)
        acc = jnp.dot(nfv, wl[0:D, :], preferred_element_type=jnp.float32)
        acc = acc + jnp.dot(agg, wl[D:2 * D, :], preferred_element_type=jnp.float32)
        out_ref[...] = jnp.maximum(acc, 0.0)

    return pl.pallas_call(
        body,
        grid=(N // CB,),
        in_specs=[
            pl.BlockSpec((1, CB, WID), lambda i: (0, i, 0)),
            pl.BlockSpec((1, CB, WID), lambda i: (1, i, 0)),
            pl.BlockSpec((CB, D), lambda i: (i, 0)),
            pl.BlockSpec((2 * D, D), lambda i: (0, 0)),
        ],
        out_specs=pl.BlockSpec((CB, D), lambda i: (i, 0)),
        out_shape=jax.ShapeDtypeStruct((N, D), jnp.float32),
    )(rst3, rst3, nf, W_lin)


def kernel(nf, edge_index, W_lin, W_attn):
    src2 = jnp.pad(edge_index[0].reshape(E // CH, CH), ((0, 2), (0, 0)))
    dst2 = jnp.pad(edge_index[1].reshape(E // CH, CH), ((0, 2), (0, 0)))
    mst = _stage_a(nf, W_attn)
    rst = _edge_stage(mst, src2, dst2)
    return _stage_c(rst, nf, W_lin)


# speedup vs baseline: 16.2877x; 1.0066x over previous
import functools

import jax
import jax.numpy as jnp
from jax import lax
from jax.experimental import pallas as pl
from jax.experimental.pallas import tpu as pltpu
from jax.experimental.pallas import tpu_sc as plsc

N = 10000
E = 160000
D = 256
WID = 144
HALF = 128
NS = 16
EPT = E // NS
CH = 100
NCH = EPT // CH
ACCR = 10240
SPT = ACCR // NS
ZB = 80
NFULL = SPT // ZB


def _stage_a(nf, W_attn):
    def body(nf_ref, wa_ref, out_ref):
        nfv = nf_ref[...]
        wa = wa_ref[0:D, 0]
        a = jnp.sum(nfv * wa[None, :], axis=1, keepdims=True)
        p = jnp.exp(a - jnp.max(a))
        out_ref[0:N, 0:HALF] = nfv[:, 0:HALF] * p
        out_ref[0:N, HALF:HALF + 1] = p
        out_ref[0:N, HALF + 1:WID] = jnp.zeros((N, WID - HALF - 1), jnp.float32)
        out_ref[N:2 * N, 0:HALF] = nfv[:, HALF:D] * p
        out_ref[N:2 * N, HALF:WID] = jnp.zeros((N, WID - HALF), jnp.float32)

    return pl.pallas_call(
        body,
        out_shape=jax.ShapeDtypeStruct((2 * N, WID), jnp.float32),
    )(nf, W_attn)


NT = NCH // 2


def _edge_stage(mst, src2, dst2):
    mesh = plsc.VectorSubcoreMesh(core_axis_name="c", subcore_axis_name="s")

    @functools.partial(
        pl.kernel,
        out_type=jax.ShapeDtypeStruct((2 * N, WID), jnp.float32),
        mesh=mesh,
        scratch_types=[
            [pltpu.VMEM((2, CH), jnp.int32)] * 2,
            [pltpu.VMEM((2, CH), jnp.int32)] * 2,
            [pltpu.VMEM((CH, WID), jnp.float32)] * 2,
            pltpu.VMEM_SHARED((ACCR, WID), jnp.float32),
            [pltpu.SemaphoreType.DMA] * 2,
            [pltpu.SemaphoreType.DMA] * 2,
            [pltpu.SemaphoreType.DMA] * 2,
            [pltpu.SemaphoreType.DMA] * 2,
        ],
        compiler_params=pltpu.CompilerParams(use_tc_tiling_on_sc=False),
    )
    def run(mst_hbm, src_hbm, dst_hbm, out_hbm, idxs, idxd, rows, acc_sh,
            semis, semid, semg, sems):
        c = lax.axis_index("c")
        s = lax.axis_index("s")
        coff = c * N
        irow0 = s * NCH

        def idx_start(t, r):
            base = irow0 + 2 * t
            pltpu.async_copy(src_hbm.at[pl.ds(base, 2)], idxs[r], semis[r])
            pltpu.async_copy(dst_hbm.at[pl.ds(base, 2)], idxd[r], semid[r])

        def idx_wait_bias(t, r):
            base = irow0 + 2 * t
            pltpu.make_async_copy(src_hbm.at[pl.ds(base, 2)], idxs[r], semis[r]).wait()
            pltpu.make_async_copy(dst_hbm.at[pl.ds(base, 2)], idxd[r], semid[r]).wait()
            for k in range(2):
                for j in range(CH // 16):
                    sl = pl.ds(j * 16, 16)
                    idxs[r][k, sl] = idxs[r][k, sl] + coff
                rem = CH - (CH // 16) * 16
                if rem:
                    sl = pl.ds(CH - 16, 16)
                    lanes = lax.iota(jnp.int32, 16)
                    idxs[r][k, sl] = idxs[r][k, sl] + jnp.where(
                        lanes >= 16 - rem, coff, 0)

        def gather_start(r, k, b):
            pltpu.async_copy(mst_hbm.at[idxs[r].at[k]], rows[b], semg[b])

        def gather_wait(r, k, b):
            pltpu.make_async_copy(mst_hbm.at[idxs[r].at[k]], rows[b], semg[b]).wait()

        def scat_start(r, k, b):
            pltpu.async_copy(rows[b], acc_sh.at[idxd[r].at[k]], sems[b], add=True)

        def scat_wait(r, k, b):
            pltpu.make_async_copy(rows[b], acc_sh.at[idxd[r].at[k]], sems[b]).wait()

        idx_start(0, 0)

        def zrow(r, carry):
            for j in range(WID // 16):
                rows[0][r, pl.ds(j * 16, 16)] = jnp.zeros((16,), jnp.float32)
            return carry

        lax.fori_loop(0, ZB, zrow, None)
        base_r = s * SPT
        for i in range(NFULL):
            pltpu.async_copy(rows[0].at[pl.ds(0, ZB)],
                             acc_sh.at[pl.ds(base_r + i * ZB, ZB)], sems[0])
        for i in range(NFULL):
            pltpu.make_async_copy(rows[0].at[pl.ds(0, ZB)],
                                  acc_sh.at[pl.ds(base_r + i * ZB, ZB)],
                                  sems[0]).wait()
        plsc.subcore_barrier()

        idx_wait_bias(0, 0)
        gather_start(0, 0, 0)

        def turn(t, r, rn):
            @pl.when(t >= 1)
            def _():
                scat_wait(r, 1, 1)

            idx_start(t + 1, rn)
            gather_start(r, 1, 1)
            gather_wait(r, 0, 0)
            scat_start(r, 0, 0)
            idx_wait_bias(t + 1, rn)
            scat_wait(r, 0, 0)
            gather_start(rn, 0, 0)
            gather_wait(r, 1, 1)
            scat_start(r, 1, 1)
            return None

        def two_turns(u, carry):
            turn(2 * u, 0, 1)
            turn(2 * u + 1, 1, 0)
            return carry

        lax.fori_loop(0, NT // 2, two_turns, None)
        scat_wait(1, 1, 1)
        gather_wait(0, 0, 0)
        plsc.subcore_barrier()

        ob = coff + base_r
        nb = jnp.where(s == NS - 1, (N - (NS - 1) * SPT) // ZB, NFULL)

        def oblk(i, carry):
            pltpu.async_copy(acc_sh.at[pl.ds(base_r + i * ZB, ZB)],
                             out_hbm.at[pl.ds(ob + i * ZB, ZB)], sems[0])
            return carry

        lax.fori_loop(0, nb, oblk, None)

        def oblk_wait(i, carry):
            pltpu.make_async_copy(acc_sh.at[pl.ds(base_r + i * ZB, ZB)],
                                  out_hbm.at[pl.ds(ob + i * ZB, ZB)],
                                  sems[0]).wait()
            return carry

        lax.fori_loop(0, nb, oblk_wait, None)

    return run(mst, src2, dst2)


CB = 1000


def _stage_c(rst, nf, W_lin):
    rst3 = rst.reshape(2, N, WID)

    def body(r0_ref, r1_ref, nf_ref, wl_ref, out_ref):
        denom = r0_ref[0, :, HALF:HALF + 1]
        ok = denom > 0.0
        dsafe = jnp.where(ok, denom, 1.0)
        agg_lo = jnp.where(ok, r0_ref[0, :, 0:HALF] / dsafe, 0.0)
        agg_hi = jnp.where(ok, r1_ref[0, :, 0:HALF] / dsafe, 0.0)
        agg = jnp.concatenate([agg_lo, agg_hi], axis=1).astype(jnp.bfloat16)
        nfv = nf_ref[...].astype(jnp.bfloat16)
        wl = wl_ref[...].astype(jnp.bfloat16)
        acc = jnp.dot(nfv, wl[0:D, :], preferred_element_type=jnp.float32)
        acc = acc + jnp.dot(agg, wl[D:2 * D, :], preferred_element_type=jnp.float32)
        out_ref[...] = jnp.maximum(acc, 0.0)

    return pl.pallas_call(
        body,
        grid=(N // CB,),
        in_specs=[
            pl.BlockSpec((1, CB, WID), lambda i: (0, i, 0)),
            pl.BlockSpec((1, CB, WID), lambda i: (1, i, 0)),
            pl.BlockSpec((CB, D), lambda i: (i, 0)),
            pl.BlockSpec((2 * D, D), lambda i: (0, 0)),
        ],
        out_specs=pl.BlockSpec((CB, D), lambda i: (i, 0)),
        out_shape=jax.ShapeDtypeStruct((N, D), jnp.float32),
    )(rst3, rst3, nf, W_lin)


def kernel(nf, edge_index, W_lin, W_attn):
    src2 = jnp.pad(edge_index[0].reshape(E // CH, CH), ((0, 2), (0, 0)))
    dst2 = jnp.pad(edge_index[1].reshape(E // CH, CH), ((0, 2), (0, 0)))
    mst = _stage_a(nf, W_attn)
    rst = _edge_stage(mst, src2, dst2)
    return _stage_c(rst, nf, W_lin)


# speedup vs baseline: 19.9389x; 1.2242x over previous
import functools

import jax
import jax.numpy as jnp
from jax import lax
from jax.experimental import pallas as pl
from jax.experimental.pallas import tpu as pltpu
from jax.experimental.pallas import tpu_sc as plsc

N = 10000
E = 160000
D = 256
WID = 128
HALF = 128
NS = 16
EPT = E // NS
CH = 100
NCH = EPT // CH
NT = NCH // 2
ACCR = 10240
SPT = ACCR // NS
ZB = 80
NFULL = SPT // ZB
DR = 10112
DZ = DR // NS


def _stage_a(nf, W_attn):
    def body(nf_ref, wa_ref, mst_ref, p_ref):
        nfv = nf_ref[...]
        wa = wa_ref[0:D, 0]
        a = jnp.sum(nfv * wa[None, :], axis=1)
        p = jnp.exp(a - jnp.max(a))
        mst_ref[0:N, :] = nfv[:, 0:HALF] * p[:, None]
        mst_ref[N:2 * N, :] = nfv[:, HALF:D] * p[:, None]
        p_ref[...] = p

    return pl.pallas_call(
        body,
        out_shape=[
            jax.ShapeDtypeStruct((2 * N, WID), jnp.float32),
            jax.ShapeDtypeStruct((N,), jnp.float32),
        ],
    )(nf, W_attn)


def _edge_stage(mst, pvec, src2, dst2):
    mesh = plsc.VectorSubcoreMesh(core_axis_name="c", subcore_axis_name="s")

    @functools.partial(
        pl.kernel,
        out_type=[
            jax.ShapeDtypeStruct((2 * N, WID), jnp.float32),
            jax.ShapeDtypeStruct((DR,), jnp.float32),
        ],
        mesh=mesh,
        scratch_types=[
            [pltpu.VMEM((2, CH), jnp.int32)] * 2,
            [pltpu.VMEM((2, CH), jnp.int32)] * 2,
            [pltpu.VMEM((CH, WID), jnp.float32)] * 2,
            [pltpu.VMEM((2, CH), jnp.float32)] * 2,
            pltpu.VMEM((N,), jnp.float32),
            pltpu.VMEM((640,), jnp.float32),
            pltpu.VMEM_SHARED((ACCR, WID), jnp.float32),
            pltpu.VMEM_SHARED((DR,), jnp.float32),
            [pltpu.SemaphoreType.DMA] * 2,
            [pltpu.SemaphoreType.DMA] * 2,
            [pltpu.SemaphoreType.DMA] * 2,
            [pltpu.SemaphoreType.DMA] * 2,
            [pltpu.SemaphoreType.DMA] * 2,
        ],
        compiler_params=pltpu.CompilerParams(use_tc_tiling_on_sc=False,
                                             needs_layout_passes=False),
    )
    def run(mst_hbm, p_hbm, src_hbm, dst_hbm, out_hbm, dout_hbm,
            idxs, idxd, rows, dval, p_v, zbuf, acc_sh, den_sh,
            semis, semid, semg, sems, semd):
        c = lax.axis_index("c")
        s = lax.axis_index("s")
        coff = c * N
        irow0 = s * NCH

        def idx_start(t, r):
            base = irow0 + 2 * t
            pltpu.async_copy(src_hbm.at[pl.ds(base, 2)], idxs[r], semis[r])
            pltpu.async_copy(dst_hbm.at[pl.ds(base, 2)], idxd[r], semid[r])

        def den_wait(r):
            for k in range(2):
                pltpu.make_async_copy(dval[r].at[k],
                                      den_sh.at[idxd[r].at[k]], semd[r]).wait()

        def idx_wait_bias(t, r, emit_denom=True):
            base = irow0 + 2 * t
            pltpu.make_async_copy(src_hbm.at[pl.ds(base, 2)], idxs[r], semis[r]).wait()
            pltpu.make_async_copy(dst_hbm.at[pl.ds(base, 2)], idxd[r], semid[r]).wait()
            lanes = lax.iota(jnp.int32, 16)
            for k in range(2):
                for j in range(CH // 16):
                    sl = pl.ds(j * 16, 16)
                    dval[r][k, sl] = plsc.load_gather(p_v, [idxs[r][k, sl]])
                rem = CH - (CH // 16) * 16
                if rem:
                    sl = pl.ds(CH - 16, 16)
                    pv = plsc.load_gather(p_v, [idxs[r][k, sl]])
                    dval[r][k, sl] = jnp.where(lanes >= 16 - rem,
                                               pv, dval[r][k, sl])
                for j in range(CH // 16):
                    sl = pl.ds(j * 16, 16)
                    idxs[r][k, sl] = idxs[r][k, sl] + coff
                if rem:
                    sl = pl.ds(CH - 16, 16)
                    idxs[r][k, sl] = idxs[r][k, sl] + jnp.where(
                        lanes >= 16 - rem, coff, 0)
            if emit_denom:
                for k in range(2):
                    pltpu.async_copy(dval[r].at[k],
                                     den_sh.at[idxd[r].at[k]], semd[r],
                                     add=True)

        def gather_start(r, k, b):
            pltpu.async_copy(mst_hbm.at[idxs[r].at[k]], rows[b], semg[b])

        def gather_wait(r, k, b):
            pltpu.make_async_copy(mst_hbm.at[idxs[r].at[k]], rows[b], semg[b]).wait()

        def scat_start(r, k, b):
            pltpu.async_copy(rows[b], acc_sh.at[idxd[r].at[k]], sems[b], add=True)

        def scat_wait(r, k, b):
            pltpu.make_async_copy(rows[b], acc_sh.at[idxd[r].at[k]], sems[b]).wait()

        idx_start(0, 0)
        pltpu.sync_copy(p_hbm, p_v)

        def zrow(i, carry):
            for j in range(WID // 16):
                rows[0][i, pl.ds(j * 16, 16)] = jnp.zeros((16,), jnp.float32)
            return carry

        lax.fori_loop(0, ZB, zrow, None)

        def zline(i, carry):
            zbuf[pl.ds(i * 16, 16)] = jnp.zeros((16,), jnp.float32)
            return carry

        lax.fori_loop(0, 640 // 16, zline, None)
        base_r = s * SPT
        for i in range(NFULL):
            pltpu.async_copy(rows[0].at[pl.ds(0, ZB)],
                             acc_sh.at[pl.ds(base_r + i * ZB, ZB)], sems[0])
        pltpu.async_copy(zbuf.at[pl.ds(0, DZ)],
                         den_sh.at[pl.ds(s * DZ, DZ)], sems[0])
        for i in range(NFULL):
            pltpu.make_async_copy(rows[0].at[pl.ds(0, ZB)],
                                  acc_sh.at[pl.ds(base_r + i * ZB, ZB)],
                                  sems[0]).wait()
        pltpu.make_async_copy(zbuf.at[pl.ds(0, DZ)],
                              den_sh.at[pl.ds(s * DZ, DZ)], sems[0]).wait()
        plsc.subcore_barrier()

        idx_wait_bias(0, 0)
        gather_start(0, 0, 0)

        def turn(t, r, rn):
            @pl.when(t >= 1)
            def _():
                scat_wait(r, 1, 1)
                den_wait(rn)

            idx_start(t + 1, rn)
            gather_start(r, 1, 1)
            gather_wait(r, 0, 0)
            scat_start(r, 0, 0)

            @pl.when(t + 1 < NT)
            def _():
                idx_wait_bias(t + 1, rn)

            @pl.when(t + 1 >= NT)
            def _():
                idx_wait_bias(t + 1, rn, emit_denom=False)

            scat_wait(r, 0, 0)
            gather_start(rn, 0, 0)
            gather_wait(r, 1, 1)
            scat_start(r, 1, 1)
            return None

        def two_turns(u, carry):
            turn(2 * u, 0, 1)
            turn(2 * u + 1, 1, 0)
            return carry

        lax.fori_loop(0, NT // 2, two_turns, None)
        scat_wait(1, 1, 1)
        den_wait(1)
        gather_wait(0, 0, 0)
        plsc.subcore_barrier()

        ob = coff + base_r
        nb = jnp.where(s == NS - 1, (N - (NS - 1) * SPT) // ZB, NFULL)

        def oblk(i, carry):
            pltpu.async_copy(acc_sh.at[pl.ds(base_r + i * ZB, ZB)],
                             out_hbm.at[pl.ds(ob + i * ZB, ZB)], sems[0])
            return carry

        lax.fori_loop(0, nb, oblk, None)

        @pl.when((c == 0) & (s == 0))
        def _():
            pltpu.sync_copy(den_sh, dout_hbm)

        def oblk_wait(i, carry):
            pltpu.make_async_copy(acc_sh.at[pl.ds(base_r + i * ZB, ZB)],
                                  out_hbm.at[pl.ds(ob + i * ZB, ZB)],
                                  sems[0]).wait()
            return carry

        lax.fori_loop(0, nb, oblk_wait, None)

    return run(mst, pvec, src2, dst2)


CB = 1000


def _stage_c(rst, denom, nf, W_lin):
    rst3 = rst.reshape(2, N, WID)

    def body(r0_ref, r1_ref, d_ref, nf_ref, wl_ref, out_ref):
        dv = d_ref[...]
        ok = dv > 0.0
        dsafe = jnp.where(ok, dv, 1.0)
        agg_lo = jnp.where(ok, r0_ref[0] / dsafe, 0.0)
        agg_hi = jnp.where(ok, r1_ref[0] / dsafe, 0.0)
        agg = jnp.concatenate([agg_lo, agg_hi], axis=1).astype(jnp.bfloat16)
        nfv = nf_ref[...].astype(jnp.bfloat16)
        wl = wl_ref[...].astype(jnp.bfloat16)
        acc = jnp.dot(nfv, wl[0:D, :], preferred_element_type=jnp.float32)
        acc = acc + jnp.dot(agg, wl[D:2 * D, :], preferred_element_type=jnp.float32)
        out_ref[...] = jnp.maximum(acc, 0.0)

    return pl.pallas_call(
        body,
        grid=(N // CB,),
        in_specs=[
            pl.BlockSpec((1, CB, WID), lambda i: (0, i, 0)),
            pl.BlockSpec((1, CB, WID), lambda i: (1, i, 0)),
            pl.BlockSpec((CB, 1), lambda i: (i, 0)),
            pl.BlockSpec((CB, D), lambda i: (i, 0)),
            pl.BlockSpec((2 * D, D), lambda i: (0, 0)),
        ],
        out_specs=pl.BlockSpec((CB, D), lambda i: (i, 0)),
        out_shape=jax.ShapeDtypeStruct((N, D), jnp.float32),
    )(rst3, rst3, denom, nf, W_lin)


def kernel(nf, edge_index, W_lin, W_attn):
    src2 = jnp.pad(edge_index[0].reshape(E // CH, CH), ((0, 2), (0, 0)))
    dst2 = jnp.pad(edge_index[1].reshape(E // CH, CH), ((0, 2), (0, 0)))
    mst, pvec = _stage_a(nf, W_attn)
    rst, dvec = _edge_stage(mst, pvec, src2, dst2)
    denom = dvec[0:N].reshape(N, 1)
    return _stage_c(rst, denom, nf, W_lin)


# speedup vs baseline: 20.2768x; 1.0169x over previous
import functools

import jax
import jax.numpy as jnp
from jax import lax
from jax.experimental import pallas as pl
from jax.experimental.pallas import tpu as pltpu
from jax.experimental.pallas import tpu_sc as plsc

N = 10000
E = 160000
D = 256
WID = 128
HALF = 128
NS = 16
EPT = E // NS
CH = 100
NCH = EPT // CH
NT = NCH // 2
ACCR = 10240
SPT = ACCR // NS
ZB = 80
NFULL = SPT // ZB
DR = 10112
DZ = DR // NS


def _stage_a(nf, W_attn):
    def body(nf_ref, wa_ref, mst_ref, p_ref):
        nfv = nf_ref[...]
        wa = wa_ref[0:D, 0]
        a = jnp.sum(nfv * wa[None, :], axis=1)
        p = jnp.exp(a - jnp.max(a))
        mst_ref[0:N, :] = nfv[:, 0:HALF] * p[:, None]
        mst_ref[N:2 * N, :] = nfv[:, HALF:D] * p[:, None]
        p_ref[...] = p

    return pl.pallas_call(
        body,
        out_shape=[
            jax.ShapeDtypeStruct((2 * N, WID), jnp.float32),
            jax.ShapeDtypeStruct((N,), jnp.float32),
        ],
    )(nf, W_attn)


def _edge_stage(mst, pvec, src2, dst2):
    mesh = plsc.VectorSubcoreMesh(core_axis_name="c", subcore_axis_name="s")

    @functools.partial(
        pl.kernel,
        out_type=[
            jax.ShapeDtypeStruct((2 * N, WID), jnp.float32),
            jax.ShapeDtypeStruct((DR,), jnp.float32),
        ],
        mesh=mesh,
        scratch_types=[
            [pltpu.VMEM((2, CH), jnp.int32)] * 2,
            [pltpu.VMEM((2, CH), jnp.int32)] * 2,
            [pltpu.VMEM((CH, WID), jnp.float32)] * 2,
            [pltpu.VMEM((2, CH), jnp.float32)] * 2,
            pltpu.VMEM((N,), jnp.float32),
            pltpu.VMEM((640,), jnp.float32),
            pltpu.VMEM_SHARED((ACCR, WID), jnp.float32),
            pltpu.VMEM_SHARED((DR,), jnp.float32),
            [pltpu.SemaphoreType.DMA] * 2,
            [pltpu.SemaphoreType.DMA] * 2,
            [pltpu.SemaphoreType.DMA] * 2,
            [pltpu.SemaphoreType.DMA] * 2,
            [pltpu.SemaphoreType.DMA] * 2,
        ],
        compiler_params=pltpu.CompilerParams(use_tc_tiling_on_sc=False,
                                             needs_layout_passes=False),
    )
    def run(mst_hbm, p_hbm, src_hbm, dst_hbm, out_hbm, dout_hbm,
            idxs, idxd, rows, dval, p_v, zbuf, acc_sh, den_sh,
            semis, semid, semg, sems, semd):
        c = lax.axis_index("c")
        s = lax.axis_index("s")
        coff = c * N
        irow0 = s * NCH

        def idx_start(t, r):
            base = irow0 + 2 * t
            pltpu.async_copy(src_hbm.at[pl.ds(base, 2)], idxs[r], semis[r])
            pltpu.async_copy(dst_hbm.at[pl.ds(base, 2)], idxd[r], semid[r])

        def den_wait(r):
            for k in range(2):
                pltpu.make_async_copy(dval[r].at[k],
                                      den_sh.at[idxd[r].at[k]], semd[r]).wait()

        def idx_wait_bias(t, r, emit_denom=True):
            base = irow0 + 2 * t
            pltpu.make_async_copy(src_hbm.at[pl.ds(base, 2)], idxs[r], semis[r]).wait()
            pltpu.make_async_copy(dst_hbm.at[pl.ds(base, 2)], idxd[r], semid[r]).wait()
            lanes = lax.iota(jnp.int32, 16)
            for k in range(2):
                for j in range(CH // 16):
                    sl = pl.ds(j * 16, 16)
                    dval[r][k, sl] = plsc.load_gather(p_v, [idxs[r][k, sl]])
                rem = CH - (CH // 16) * 16
                if rem:
                    sl = pl.ds(CH - 16, 16)
                    pv = plsc.load_gather(p_v, [idxs[r][k, sl]])
                    dval[r][k, sl] = jnp.where(lanes >= 16 - rem,
                                               pv, dval[r][k, sl])
                for j in range(CH // 16):
                    sl = pl.ds(j * 16, 16)
                    idxs[r][k, sl] = idxs[r][k, sl] + coff
                if rem:
                    sl = pl.ds(CH - 16, 16)
                    idxs[r][k, sl] = idxs[r][k, sl] + jnp.where(
                        lanes >= 16 - rem, coff, 0)
            if emit_denom:
                for k in range(2):
                    pltpu.async_copy(dval[r].at[k],
                                     den_sh.at[idxd[r].at[k]], semd[r],
                                     add=True)

        def gather_start(r, k, b):
            pltpu.async_copy(mst_hbm.at[idxs[r].at[k]], rows[b], semg[b])

        def gather_wait(r, k, b):
            pltpu.make_async_copy(mst_hbm.at[idxs[r].at[k]], rows[b], semg[b]).wait()

        def scat_start(r, k, b):
            pltpu.async_copy(rows[b], acc_sh.at[idxd[r].at[k]], sems[b], add=True)

        def scat_wait(r, k, b):
            pltpu.make_async_copy(rows[b], acc_sh.at[idxd[r].at[k]], sems[b]).wait()

        idx_start(0, 0)
        pltpu.async_copy(p_hbm, p_v, semg[1])

        def zrow(i, carry):
            for j in range(WID // 16):
                rows[0][i, pl.ds(j * 16, 16)] = jnp.zeros((16,), jnp.float32)
            return carry

        lax.fori_loop(0, ZB, zrow, None)

        def zline(i, carry):
            zbuf[pl.ds(i * 16, 16)] = jnp.zeros((16,), jnp.float32)
            return carry

        lax.fori_loop(0, 640 // 16, zline, None)
        base_r = s * SPT
        for i in range(NFULL):
            pltpu.async_copy(rows[0].at[pl.ds(0, ZB)],
                             acc_sh.at[pl.ds(base_r + i * ZB, ZB)], sems[0])
        pltpu.async_copy(zbuf.at[pl.ds(0, DZ)],
                         den_sh.at[pl.ds(s * DZ, DZ)], sems[0])
        for i in range(NFULL):
            pltpu.make_async_copy(rows[0].at[pl.ds(0, ZB)],
                                  acc_sh.at[pl.ds(base_r + i * ZB, ZB)],
                                  sems[0]).wait()
        pltpu.make_async_copy(zbuf.at[pl.ds(0, DZ)],
                              den_sh.at[pl.ds(s * DZ, DZ)], sems[0]).wait()
        pltpu.make_async_copy(p_hbm, p_v, semg[1]).wait()
        plsc.subcore_barrier()

        idx_wait_bias(0, 0)
        gather_start(0, 0, 0)

        def turn(t, r, rn):
            @pl.when(t >= 1)
            def _():
                scat_wait(r, 1, 1)
                den_wait(rn)

            idx_start(t + 1, rn)
            gather_start(r, 1, 1)
            gather_wait(r, 0, 0)
            scat_start(r, 0, 0)

            @pl.when(t + 1 < NT)
            def _():
                idx_wait_bias(t + 1, rn)

            @pl.when(t + 1 >= NT)
            def _():
                idx_wait_bias(t + 1, rn, emit_denom=False)

            scat_wait(r, 0, 0)
            gather_start(rn, 0, 0)
            gather_wait(r, 1, 1)
            scat_start(r, 1, 1)
            return None

        def two_turns(u, carry):
            turn(2 * u, 0, 1)
            turn(2 * u + 1, 1, 0)
            return carry

        lax.fori_loop(0, NT // 2, two_turns, None)
        scat_wait(1, 1, 1)
        den_wait(1)
        gather_wait(0, 0, 0)
        plsc.subcore_barrier()

        ob = coff + base_r
        nb = jnp.where(s == NS - 1, (N - (NS - 1) * SPT) // ZB, NFULL)

        def oblk(i, carry):
            pltpu.async_copy(acc_sh.at[pl.ds(base_r + i * ZB, ZB)],
                             out_hbm.at[pl.ds(ob + i * ZB, ZB)], sems[0])
            return carry

        lax.fori_loop(0, nb, oblk, None)

        @pl.when((c == 0) & (s == 0))
        def _():
            pltpu.sync_copy(den_sh, dout_hbm)

        def oblk_wait(i, carry):
            pltpu.make_async_copy(acc_sh.at[pl.ds(base_r + i * ZB, ZB)],
                                  out_hbm.at[pl.ds(ob + i * ZB, ZB)],
                                  sems[0]).wait()
            return carry

        lax.fori_loop(0, nb, oblk_wait, None)

    return run(mst, pvec, src2, dst2)


CB = 1000


def _stage_c(rst, denom, nf, W_lin):
    rst3 = rst.reshape(2, N, WID)

    def body(r0_ref, r1_ref, d_ref, nf_ref, wl_ref, out_ref):
        dv = d_ref[...]
        ok = dv > 0.0
        dsafe = jnp.where(ok, dv, 1.0)
        agg_lo = jnp.where(ok, r0_ref[0] / dsafe, 0.0)
        agg_hi = jnp.where(ok, r1_ref[0] / dsafe, 0.0)
        agg = jnp.concatenate([agg_lo, agg_hi], axis=1).astype(jnp.bfloat16)
        nfv = nf_ref[...].astype(jnp.bfloat16)
        wl = wl_ref[...].astype(jnp.bfloat16)
        acc = jnp.dot(nfv, wl[0:D, :], preferred_element_type=jnp.float32)
        acc = acc + jnp.dot(agg, wl[D:2 * D, :], preferred_element_type=jnp.float32)
        out_ref[...] = jnp.maximum(acc, 0.0)

    return pl.pallas_call(
        body,
        grid=(N // CB,),
        in_specs=[
            pl.BlockSpec((1, CB, WID), lambda i: (0, i, 0)),
            pl.BlockSpec((1, CB, WID), lambda i: (1, i, 0)),
            pl.BlockSpec((CB, 1), lambda i: (i, 0)),
            pl.BlockSpec((CB, D), lambda i: (i, 0)),
            pl.BlockSpec((2 * D, D), lambda i: (0, 0)),
        ],
        out_specs=pl.BlockSpec((CB, D), lambda i: (i, 0)),
        out_shape=jax.ShapeDtypeStruct((N, D), jnp.float32),
    )(rst3, rst3, denom, nf, W_lin)


def kernel(nf, edge_index, W_lin, W_attn):
    src2 = jnp.pad(edge_index[0].reshape(E // CH, CH), ((0, 2), (0, 0)))
    dst2 = jnp.pad(edge_index[1].reshape(E // CH, CH), ((0, 2), (0, 0)))
    mst, pvec = _stage_a(nf, W_attn)
    rst, dvec = _edge_stage(mst, pvec, src2, dst2)
    denom = dvec[0:N].reshape(N, 1)
    return _stage_c(rst, denom, nf, W_lin)
